# baseline (speedup 1.0000x reference)
import functools

import jax
import jax.numpy as jnp
from jax import lax
from jax.experimental import pallas as pl
from jax.experimental.pallas import tpu as pltpu
from jax.experimental.pallas import tpu_sc as plsc

_NC = 2
_NS = 16
_NW = _NC * _NS
_G = 16


def _sc_sum_pool(x_flat, emb, b_total, seq_pad):
    d = emb.shape[1]
    nv = d // 16
    bpw = b_total // _NW
    spr = seq_pad // _G
    tot = bpw * spr
    mesh = plsc.VectorSubcoreMesh(core_axis_name="c", subcore_axis_name="s")

    nbuf = 16
    @functools.partial(
        pl.kernel,
        out_type=jax.ShapeDtypeStruct((b_total, d), jnp.float32),
        mesh=mesh,
        compiler_params=pltpu.CompilerParams(use_tc_tiling_on_sc=False),
        scratch_types=(
            [pltpu.VMEM((tot * _G,), jnp.int32)]
            + [pltpu.VMEM((_G, d), jnp.float32) for _ in range(nbuf)]
            + [pltpu.VMEM((bpw, d), jnp.float32)]
            + [pltpu.SemaphoreType.DMA for _ in range(nbuf)]
        ),
    )
    def pool(x_hbm, emb_hbm, out_hbm, idx_v, *refs):
        bufs = refs[:nbuf]
        zbuf = refs[nbuf]
        gsems = refs[nbuf + 1:]

        s = lax.axis_index("s")
        c = lax.axis_index("c")
        wid = s * _NC + c
        pltpu.sync_copy(x_hbm.at[pl.ds(wid * tot * _G, tot * _G)], idx_v)

        def issue(g, j):
            ivec = idx_v[pl.ds(g * _G, _G)]
            pltpu.async_copy(emb_hbm.at[ivec], bufs[j], gsems[j])

        def gdrain(j):
            pltpu.make_async_copy(emb_hbm.at[idx_v[pl.ds(0, _G)]],
                                  bufs[j], gsems[j]).wait()

        for j in range(nbuf):
            issue(jnp.int32(j), j)

        zv = jnp.zeros((16,), jnp.float32)
        last = jnp.int32(tot - 1)

        def body(i, carry):
            row, phase = carry[0], carry[1]
            accs = list(carry[2:])
            for j in range(nbuf):
                g = i * nbuf + j
                gdrain(j)
                for t in range(_G):
                    for w in range(nv):
                        accs[w] = accs[w] + bufs[j][t, pl.ds(w * 16, 16)]
                issue(jnp.minimum(g + nbuf, last), j)
                for w in range(nv):
                    zbuf[row, pl.ds(w * 16, 16)] = accs[w]
                wrap = phase == (spr - 1)
                for w in range(nv):
                    accs[w] = jnp.where(wrap, zv, accs[w])
                row = row + wrap.astype(jnp.int32)
                phase = jnp.where(wrap, 0, phase + 1)
            return (row, phase, *accs)

        lax.fori_loop(0, tot // nbuf, body,
                      (jnp.int32(0), jnp.int32(0)) + (zv,) * nv)
        for j in range(nbuf):
            gdrain(j)
        pltpu.sync_copy(zbuf, out_hbm.at[pl.ds(wid * bpw, bpw)])

    return pool(x_flat, emb)


def _tc_mlp(sums, x, row0, W1, b1, W2p, b2p, seq_pad):
    b_total, _ = sums.shape
    h = W1.shape[1]
    o = W2p.shape[1]

    def body(s_ref, x_ref, r0_ref, w1_ref, b1_ref, w2_ref, b2_ref, o_ref):
        lenf = jnp.sum((x_ref[...] != 0).astype(jnp.float32), axis=1,
                       keepdims=True)
        pad_cnt = seq_pad - lenf
        avg = (s_ref[...] - pad_cnt * r0_ref[...]) / jnp.maximum(lenf, 1.0)
        hh = jnp.dot(avg, w1_ref[...], preferred_element_type=jnp.float32)
        hh = jnp.maximum(hh + b1_ref[...], 0.0)
        o_ref[...] = jnp.dot(hh, w2_ref[...],
                             preferred_element_type=jnp.float32) + b2_ref[...]

    return pl.pallas_call(
        body,
        out_shape=jax.ShapeDtypeStruct((b_total, o), jnp.float32),
    )(sums, x, row0, W1, b1.reshape(1, h), W2p, b2p.reshape(1, o))


def kernel(x, emb_table, W1, b1, W2, b2):
    x = x.astype(jnp.int32)
    b_total, s = x.shape
    spr = -(-s // _G)
    seq_pad = spr * _G
    x_flat = jnp.pad(x, ((0, 0), (0, seq_pad - s))).reshape(-1)
    sums = _sc_sum_pool(x_flat, emb_table, b_total, seq_pad)
    o = 128
    w2p = jnp.pad(W2, ((0, 0), (0, o - W2.shape[1])))
    b2p = jnp.pad(b2, (0, o - b2.shape[0]))
    row0 = emb_table[0:1]
    out = _tc_mlp(sums, x, row0, W1, b1, w2p, b2p, float(seq_pad))
    return out[:, : W2.shape[1]]


# speedup vs baseline: 1.0120x; 1.0120x over previous
import functools

import jax
import jax.numpy as jnp
from jax import lax
from jax.experimental import pallas as pl
from jax.experimental.pallas import tpu as pltpu
from jax.experimental.pallas import tpu_sc as plsc

_NC = 2
_NS = 16
_NW = _NC * _NS
_G = 16


def _sc_sum_pool(x_flat, emb, b_total, seq_pad):
    d = emb.shape[1]
    nv = d // 16
    bpw = b_total // _NW
    spr = seq_pad // _G
    tot = bpw * spr
    mesh = plsc.VectorSubcoreMesh(core_axis_name="c", subcore_axis_name="s")

    @functools.partial(
        pl.kernel,
        out_type=jax.ShapeDtypeStruct((b_total, d), jnp.float32),
        mesh=mesh,
        compiler_params=pltpu.CompilerParams(use_tc_tiling_on_sc=False),
        scratch_types=(
            [pltpu.VMEM((tot * _G,), jnp.int32)]
            + [pltpu.VMEM((_G, d), jnp.float32) for _ in range(spr)]
            + [pltpu.VMEM((bpw, d), jnp.float32)]
            + [pltpu.SemaphoreType.DMA for _ in range(spr)]
        ),
    )
    def pool(x_hbm, emb_hbm, out_hbm, idx_v, *refs):
        bufs = refs[:spr]
        zbuf = refs[spr]
        gsems = refs[spr + 1:]

        s = lax.axis_index("s")
        c = lax.axis_index("c")
        wid = s * _NC + c
        pltpu.sync_copy(x_hbm.at[pl.ds(wid * tot * _G, tot * _G)], idx_v)

        def issue(g, j):
            ivec = idx_v[pl.ds(g * _G, _G)]
            pltpu.async_copy(emb_hbm.at[ivec], bufs[j], gsems[j])

        def gdrain(j):
            pltpu.make_async_copy(emb_hbm.at[idx_v[pl.ds(0, _G)]],
                                  bufs[j], gsems[j]).wait()

        for j in range(spr):
            issue(jnp.int32(j), j)

        zv = jnp.zeros((16,), jnp.float32)
        last = jnp.int32(tot - 1)

        def body(r, carry):
            accs = [zv] * nv
            for j in range(spr):
                gdrain(j)
                for t in range(_G):
                    for w in range(nv):
                        accs[w] = accs[w] + bufs[j][t, pl.ds(w * 16, 16)]
                issue(jnp.minimum((r + 1) * spr + j, last), j)
            for w in range(nv):
                zbuf[r, pl.ds(w * 16, 16)] = accs[w]
            return carry

        lax.fori_loop(0, bpw, body, jnp.int32(0))
        for j in range(spr):
            gdrain(j)
        pltpu.sync_copy(zbuf, out_hbm.at[pl.ds(wid * bpw, bpw)])

    return pool(x_flat, emb)


def _tc_mlp(sums, x, row0, W1, b1, W2p, b2p, seq_pad):
    b_total, _ = sums.shape
    h = W1.shape[1]
    o = W2p.shape[1]

    def body(s_ref, x_ref, r0_ref, w1_ref, b1_ref, w2_ref, b2_ref, o_ref):
        lenf = jnp.sum((x_ref[...] != 0).astype(jnp.float32), axis=1,
                       keepdims=True)
        pad_cnt = seq_pad - lenf
        avg = (s_ref[...] - pad_cnt * r0_ref[...]) / jnp.maximum(lenf, 1.0)
        hh = jnp.dot(avg, w1_ref[...], preferred_element_type=jnp.float32)
        hh = jnp.maximum(hh + b1_ref[...], 0.0)
        o_ref[...] = jnp.dot(hh, w2_ref[...],
                             preferred_element_type=jnp.float32) + b2_ref[...]

    return pl.pallas_call(
        body,
        out_shape=jax.ShapeDtypeStruct((b_total, o), jnp.float32),
    )(sums, x, row0, W1, b1.reshape(1, h), W2p, b2p.reshape(1, o))


def kernel(x, emb_table, W1, b1, W2, b2):
    x = x.astype(jnp.int32)
    b_total, s = x.shape
    spr = -(-s // _G)
    seq_pad = spr * _G
    x_flat = jnp.pad(x, ((0, 0), (0, seq_pad - s))).reshape(-1)
    sums = _sc_sum_pool(x_flat, emb_table, b_total, seq_pad)
    o = 128
    w2p = jnp.pad(W2, ((0, 0), (0, o - W2.shape[1])))
    b2p = jnp.pad(b2, (0, o - b2.shape[0]))
    row0 = emb_table[0:1]
    out = _tc_mlp(sums, x, row0, W1, b1, w2p, b2p, float(seq_pad))
    return out[:, : W2.shape[1]]
